# baseline (speedup 1.0000x reference)
import jax
import jax.numpy as jnp
from jax.experimental import pallas as pl
from jax.experimental.pallas import tpu as pltpu

NUM_EMB = 1024
EMB_DIM = 256
COMMIT = 0.25
DECAY = 0.99
EPS = 1e-05

TILE_N = 1024


def _pass1_body(x_ref, e_ref, dist_ref, idx2_ref, counts_ref, dw_ref, xn2_ref,
                en_scratch):
    j = pl.program_id(0)
    b = j // 2
    h = j % 2

    @pl.when(j == 0)
    def _norms():
        e0 = e_ref[...]
        en_scratch[0, :] = jnp.sum(e0 * e0, axis=1)

    x = x_ref[:, b, pl.ds(h * TILE_N, TILE_N)]
    e = e_ref[...]
    cross = jax.lax.dot_general(
        x, e, (((0,), (1,)), ((), ())), preferred_element_type=jnp.float32)
    xn = jnp.sum(x * x, axis=0)
    en = en_scratch[0, :]
    dist = xn[:, None] + en[None, :] - 2.0 * cross
    dist_ref[...] = dist
    d3 = dist.reshape(TILE_N // 128, 128, NUM_EMB)
    iota3 = jax.lax.broadcasted_iota(jnp.int32, d3.shape, 2)
    idx2 = jnp.argmin(d3, axis=2).astype(jnp.int32)
    idx2_ref[...] = idx2
    enc = (iota3 == idx2[:, :, None]).astype(jnp.float32).reshape(
        TILE_N, NUM_EMB)
    part_counts = jax.lax.dot_general(
        jnp.ones((1, TILE_N), jnp.float32), enc, (((1,), (0,)), ((), ())),
        preferred_element_type=jnp.float32)
    part_dw = jax.lax.dot_general(
        enc, x, (((0,), (1,)), ((), ())), preferred_element_type=jnp.float32)

    @pl.when(j == 0)
    def _init():
        counts_ref[...] = part_counts
        dw_ref[...] = part_dw
        xn2_ref[0, 0] = jnp.sum(xn)

    @pl.when(j != 0)
    def _acc():
        counts_ref[...] += part_counts
        dw_ref[...] += part_dw
        xn2_ref[0, 0] += jnp.sum(xn)


def _pass2_body(idx2_ref, counts_ref, dw_ref, ema_w_ref, ecs_ref, xn2_ref,
                outq_ref, loss_ref, ppl_ref, nw_scratch):
    j = pl.program_id(0)
    nsteps = pl.num_programs(0)
    n_total = jnp.float32(nsteps * TILE_N)

    @pl.when(j == 0)
    def _setup():
        counts = counts_ref[0, :]
        t = ecs_ref[0, :] * DECAY + (1.0 - DECAY) * counts
        n = jnp.sum(t)
        t = (t + EPS) / (n + NUM_EMB * EPS) * n
        dwv = dw_ref[...]
        nw = (ema_w_ref[...] * DECAY + (1.0 - DECAY) * dwv) / t[:, None]
        nw_scratch[...] = nw
        p = counts / n_total
        ppl_ref[0, 0] = jnp.exp(-jnp.sum(p * jnp.log(p + 1e-10)))
        sse = (jnp.sum(jnp.sum(nw * nw, axis=1) * counts)
               - 2.0 * jnp.sum(nw * dwv) + xn2_ref[0, 0])
        loss_ref[0, 0] = COMMIT * sse / (n_total * EMB_DIM)

    b = j // 2
    h = j % 2
    idx2 = idx2_ref[...]
    iota3 = jax.lax.broadcasted_iota(
        jnp.int32, (TILE_N // 128, 128, NUM_EMB), 2)
    enc = (iota3 == idx2[:, :, None]).astype(jnp.float32).reshape(
        TILE_N, NUM_EMB)
    q = jax.lax.dot_general(
        nw_scratch[...], enc, (((0,), (1,)), ((), ())),
        preferred_element_type=jnp.float32)
    outq_ref[:, b, pl.ds(h * TILE_N, TILE_N)] = q


def kernel(inputs, embedding_weight, ema_w, ema_cluster_size):
    D, B, T = inputs.shape
    N = B * T
    K = embedding_weight.shape[0]
    nt = N // TILE_N
    L = N * K // (D * T)

    dist, idx2, counts, dw, xn2 = pl.pallas_call(
        _pass1_body,
        grid=(nt,),
        in_specs=[
            pl.BlockSpec((D, B, T), lambda j: (0, 0, 0)),
            pl.BlockSpec((K, D), lambda j: (0, 0)),
        ],
        out_specs=[
            pl.BlockSpec((TILE_N, K), lambda j: (j, 0)),
            pl.BlockSpec((TILE_N // 128, 128), lambda j: (j, 0)),
            pl.BlockSpec((1, K), lambda j: (0, 0)),
            pl.BlockSpec((K, D), lambda j: (0, 0)),
            pl.BlockSpec(memory_space=pltpu.SMEM),
        ],
        out_shape=[
            jax.ShapeDtypeStruct((N, K), jnp.float32),
            jax.ShapeDtypeStruct((N // 128, 128), jnp.int32),
            jax.ShapeDtypeStruct((1, K), jnp.float32),
            jax.ShapeDtypeStruct((K, D), jnp.float32),
            jax.ShapeDtypeStruct((1, 1), jnp.float32),
        ],
        scratch_shapes=[
            pltpu.VMEM((1, K), jnp.float32),
        ],
    )(inputs, embedding_weight)

    outq, loss, ppl = pl.pallas_call(
        _pass2_body,
        grid=(nt,),
        in_specs=[
            pl.BlockSpec((TILE_N // 128, 128), lambda j: (j, 0)),
            pl.BlockSpec((1, K), lambda j: (0, 0)),
            pl.BlockSpec((K, D), lambda j: (0, 0)),
            pl.BlockSpec((K, D), lambda j: (0, 0)),
            pl.BlockSpec((1, K), lambda j: (0, 0)),
            pl.BlockSpec(memory_space=pltpu.SMEM),
        ],
        out_specs=[
            pl.BlockSpec((D, B, T), lambda j: (0, 0, 0)),
            pl.BlockSpec(memory_space=pltpu.SMEM),
            pl.BlockSpec(memory_space=pltpu.SMEM),
        ],
        out_shape=[
            jax.ShapeDtypeStruct((D, B, T), jnp.float32),
            jax.ShapeDtypeStruct((1, 1), jnp.float32),
            jax.ShapeDtypeStruct((1, 1), jnp.float32),
        ],
        scratch_shapes=[
            pltpu.VMEM((K, D), jnp.float32),
        ],
    )(idx2, counts, dw, ema_w, ema_cluster_size.reshape(1, K), xn2)

    g = T // (N // D)
    rep = jnp.repeat(idx2, g, axis=1)
    kk = (jnp.arange(T, dtype=jnp.int32)[:, None] % g) * L + jnp.arange(L, dtype=jnp.int32)[None, :]
    enc3 = (rep[:, :, None] == kk[None, :, :]).astype(jnp.float32)
    return (loss[0, 0], outq, ppl[0, 0], enc3,
            dist.reshape(D, T, L), idx2.reshape(N)[:, None])


# speedup vs baseline: 1.0146x; 1.0146x over previous
import jax
import jax.numpy as jnp
from jax.experimental import pallas as pl
from jax.experimental.pallas import tpu as pltpu

NUM_EMB = 1024
EMB_DIM = 256
COMMIT = 0.25
DECAY = 0.99
EPS = 1e-05

TILE_N = 1024


def _pass1_body(x_ref, e_ref, dist_ref, idx2_ref, counts_ref, dw_ref, xn2_ref):
    j = pl.program_id(0)
    b = j // 2
    h = j % 2
    x = x_ref[:, b, pl.ds(h * TILE_N, TILE_N)]
    e = e_ref[...]
    cross = jax.lax.dot_general(
        x, e, (((0,), (1,)), ((), ())), preferred_element_type=jnp.float32)
    xn = jnp.sum(x * x, axis=0)
    en = jnp.sum(e * e, axis=1)
    dist = xn[:, None] + en[None, :] - 2.0 * cross
    dist_ref[...] = dist
    d3 = dist.reshape(TILE_N // 128, 128, NUM_EMB)
    iota3 = jax.lax.broadcasted_iota(jnp.int32, d3.shape, 2)
    idx2 = jnp.argmin(d3, axis=2).astype(jnp.int32)
    idx2_ref[...] = idx2
    enc = (iota3 == idx2[:, :, None]).astype(jnp.float32).reshape(
        TILE_N, NUM_EMB)
    part_counts = jnp.sum(enc, axis=0)[None, :]
    part_dw = jax.lax.dot_general(
        enc, x, (((0,), (1,)), ((), ())), preferred_element_type=jnp.float32)

    @pl.when(j == 0)
    def _init():
        counts_ref[...] = part_counts
        dw_ref[...] = part_dw
        xn2_ref[0, 0] = jnp.sum(xn)

    @pl.when(j != 0)
    def _acc():
        counts_ref[...] += part_counts
        dw_ref[...] += part_dw
        xn2_ref[0, 0] += jnp.sum(xn)


def _pass2_body(idx2_ref, counts_ref, dw_ref, ema_w_ref, ecs_ref, xn2_ref,
                outq_ref, loss_ref, ppl_ref, nw_scratch):
    j = pl.program_id(0)
    nsteps = pl.num_programs(0)
    n_total = jnp.float32(nsteps * TILE_N)

    @pl.when(j == 0)
    def _setup():
        counts = counts_ref[0, :]
        t = ecs_ref[0, :] * DECAY + (1.0 - DECAY) * counts
        n = jnp.sum(t)
        t = (t + EPS) / (n + NUM_EMB * EPS) * n
        dwv = dw_ref[...]
        nw = (ema_w_ref[...] * DECAY + (1.0 - DECAY) * dwv) / t[:, None]
        nw_scratch[...] = nw
        p = counts / n_total
        ppl_ref[0, 0] = jnp.exp(-jnp.sum(p * jnp.log(p + 1e-10)))
        sse = (jnp.sum(jnp.sum(nw * nw, axis=1) * counts)
               - 2.0 * jnp.sum(nw * dwv) + xn2_ref[0, 0])
        loss_ref[0, 0] = COMMIT * sse / (n_total * EMB_DIM)

    b = j // 2
    h = j % 2
    idx2 = idx2_ref[...]
    iota3 = jax.lax.broadcasted_iota(
        jnp.int32, (TILE_N // 128, 128, NUM_EMB), 2)
    enc = (iota3 == idx2[:, :, None]).astype(jnp.float32).reshape(
        TILE_N, NUM_EMB)
    q = jax.lax.dot_general(
        nw_scratch[...], enc, (((0,), (1,)), ((), ())),
        preferred_element_type=jnp.float32)
    outq_ref[:, b, pl.ds(h * TILE_N, TILE_N)] = q


def kernel(inputs, embedding_weight, ema_w, ema_cluster_size):
    D, B, T = inputs.shape
    N = B * T
    K = embedding_weight.shape[0]
    nt = N // TILE_N
    L = N * K // (D * T)

    dist, idx2, counts, dw, xn2 = pl.pallas_call(
        _pass1_body,
        grid=(nt,),
        in_specs=[
            pl.BlockSpec((D, B, T), lambda j: (0, 0, 0)),
            pl.BlockSpec((K, D), lambda j: (0, 0)),
        ],
        out_specs=[
            pl.BlockSpec((TILE_N, K), lambda j: (j, 0)),
            pl.BlockSpec((TILE_N // 128, 128), lambda j: (j, 0)),
            pl.BlockSpec((1, K), lambda j: (0, 0)),
            pl.BlockSpec((K, D), lambda j: (0, 0)),
            pl.BlockSpec(memory_space=pltpu.SMEM),
        ],
        out_shape=[
            jax.ShapeDtypeStruct((N, K), jnp.float32),
            jax.ShapeDtypeStruct((N // 128, 128), jnp.int32),
            jax.ShapeDtypeStruct((1, K), jnp.float32),
            jax.ShapeDtypeStruct((K, D), jnp.float32),
            jax.ShapeDtypeStruct((1, 1), jnp.float32),
        ],
    )(inputs, embedding_weight)

    outq, loss, ppl = pl.pallas_call(
        _pass2_body,
        grid=(nt,),
        in_specs=[
            pl.BlockSpec((TILE_N // 128, 128), lambda j: (j, 0)),
            pl.BlockSpec((1, K), lambda j: (0, 0)),
            pl.BlockSpec((K, D), lambda j: (0, 0)),
            pl.BlockSpec((K, D), lambda j: (0, 0)),
            pl.BlockSpec((1, K), lambda j: (0, 0)),
            pl.BlockSpec(memory_space=pltpu.SMEM),
        ],
        out_specs=[
            pl.BlockSpec((D, B, T), lambda j: (0, 0, 0)),
            pl.BlockSpec(memory_space=pltpu.SMEM),
            pl.BlockSpec(memory_space=pltpu.SMEM),
        ],
        out_shape=[
            jax.ShapeDtypeStruct((D, B, T), jnp.float32),
            jax.ShapeDtypeStruct((1, 1), jnp.float32),
            jax.ShapeDtypeStruct((1, 1), jnp.float32),
        ],
        scratch_shapes=[
            pltpu.VMEM((K, D), jnp.float32),
        ],
    )(idx2, counts, dw, ema_w, ema_cluster_size.reshape(1, K), xn2)

    g = T // (N // D)
    rep = jnp.repeat(idx2, g, axis=1)
    kk = (jnp.arange(T, dtype=jnp.int32)[:, None] % g) * L + jnp.arange(L, dtype=jnp.int32)[None, :]
    enc3 = (rep[:, :, None] == kk[None, :, :]).astype(jnp.float32)
    return (loss[0, 0], outq, ppl[0, 0], enc3,
            dist.reshape(D, T, L), idx2.reshape(N)[:, None])
